# baseline (speedup 1.0000x reference)
import jax
import jax.numpy as jnp
from jax import lax
from jax.experimental import pallas as pl
from jax.experimental.pallas import tpu as pltpu
from jax.experimental.pallas import tpu_sc as plsc

SEQ = 200
DIM = 64
BATCH = 4096
NUM_CORES = 2
NUM_SUBCORES = 16
NW = NUM_CORES * NUM_SUBCORES
BBLK = BATCH // NW
NBUF = 4
LANES = 16
SCALE = 8.0


def _body(inp_ref, tok_ref, pos_ref, out_ref,
          idx_v, pos_v, rows0, rows1, rows2, rows3, ob0, ob1, ob2, ob3,
          gsem0, gsem1, gsem2, gsem3, ssem0, ssem1, ssem2, ssem3):
  rows = (rows0, rows1, rows2, rows3)
  obuf = (ob0, ob1, ob2, ob3)
  gsem = (gsem0, gsem1, gsem2, gsem3)
  ssem = (ssem0, ssem1, ssem2, ssem3)

  w = lax.axis_index("s") * NUM_CORES + lax.axis_index("c")

  def start_gather(j, s):
    sh = s // 8
    sl = s - sh * 8
    pltpu.async_copy(tok_ref.at[idx_v.at[sh, sl]], rows[j], gsem[j])

  def wait_gather(j):
    pltpu.make_async_copy(tok_ref.at[pl.ds(0, BBLK)], rows[j], gsem[j]).wait()

  def start_scatter(j, s):
    pltpu.async_copy(obuf[j], out_ref.at[s, :, w], ssem[j])

  def wait_scatter(j):
    pltpu.make_async_copy(obuf[j], out_ref.at[0, :, w], ssem[j]).wait()

  iota = lax.iota(jnp.int32, LANES)
  idxb = [iota + bq * LANES for bq in range(BBLK // LANES)]

  def compute(j, s):
    idx_s = jnp.broadcast_to(s, (LANES,))

    @pl.loop(0, DIM)
    def _(d):
      dh = d // 8
      dl = d - dh * 8
      idx_d = jnp.broadcast_to(d, (LANES,))
      p = plsc.load_gather(pos_v, [idx_s, idx_d])
      for bq in range(BBLK // LANES):
        v = plsc.load_gather(rows[j], [idxb[bq], idx_d])
        obuf[j][dh, pl.ds(dl * 128 + bq * LANES, LANES)] = v * SCALE + p

  for sh in range(SEQ // 8):
    pltpu.sync_copy(inp_ref.at[sh, w], idx_v.at[sh])
  pltpu.sync_copy(pos_ref, pos_v)

  for j in range(NBUF):
    start_gather(j, jnp.int32(j))

  @pl.loop(0, SEQ // NBUF)
  def _(grp):
    for j in range(NBUF):
      s = grp * NBUF + j
      wait_gather(j)

      @pl.when(s >= NBUF)
      def _():
        wait_scatter(j)

      compute(j, s)
      start_scatter(j, s)

      @pl.when(s + NBUF < SEQ)
      def _():
        start_gather(j, s + NBUF)

  for j in range(NBUF):
    wait_scatter(j)


@jax.jit
def _embed(inp4d, token_table, pos_table):
  mesh = plsc.VectorSubcoreMesh(core_axis_name="c", subcore_axis_name="s")
  run = pl.kernel(
      _body,
      out_type=jax.ShapeDtypeStruct((SEQ, DIM // 8, NW, 8 * 128), jnp.float32),
      mesh=mesh,
      compiler_params=pltpu.CompilerParams(
          use_tc_tiling_on_sc=False, needs_layout_passes=False),
      scratch_types=[
          pltpu.VMEM((SEQ // 8, 8, 128), jnp.int32),
          pltpu.VMEM((SEQ, DIM), jnp.float32),
      ] + [pltpu.VMEM((BBLK, DIM), jnp.float32) for _ in range(NBUF)]
        + [pltpu.VMEM((DIM // 8, 8 * 128), jnp.float32) for _ in range(NBUF)]
        + [pltpu.SemaphoreType.DMA for _ in range(2 * NBUF)],
  )
  return run(inp4d, token_table, pos_table)


def kernel(inputs, token_table, pos_table):
  inp4d = (inputs.astype(jnp.int32).T
           .reshape(SEQ // 8, 8, NW, BBLK)
           .transpose(0, 2, 1, 3))
  out4d = _embed(inp4d, token_table, pos_table)
  out = (out4d.reshape(SEQ, DIM // 8, NW, 8, BBLK)
         .transpose(2, 4, 0, 1, 3)
         .reshape(BATCH, SEQ, DIM))
  return out


# speedup vs baseline: 1.6762x; 1.6762x over previous
import jax
import jax.numpy as jnp
from jax import lax
from jax.experimental import pallas as pl
from jax.experimental.pallas import tpu as pltpu
from jax.experimental.pallas import tpu_sc as plsc

SEQ = 200
DIM = 64
BATCH = 4096
NUM_CORES = 2
NUM_SUBCORES = 16
NW = NUM_CORES * NUM_SUBCORES
BBLK = BATCH // NW
NBUF = 4
LANES = 16
NQ = DIM // LANES
SCALE = 8.0


def _body(inp_ref, tok_ref, pos_ref, out_ref,
          idx_v, pos_v, rows0, rows1, rows2, rows3, ob0, ob1, ob2, ob3,
          gsem0, gsem1, gsem2, gsem3, ssem0, ssem1, ssem2, ssem3):
  rows = (rows0, rows1, rows2, rows3)
  obuf = (ob0, ob1, ob2, ob3)
  gsem = (gsem0, gsem1, gsem2, gsem3)
  ssem = (ssem0, ssem1, ssem2, ssem3)

  w = lax.axis_index("s") * NUM_CORES + lax.axis_index("c")

  def start_gather(j, s):
    sh = s // 8
    sl = s - sh * 8
    pltpu.async_copy(tok_ref.at[idx_v.at[sh, sl]], rows[j], gsem[j])

  def wait_gather(j):
    pltpu.make_async_copy(tok_ref.at[pl.ds(0, BBLK)], rows[j], gsem[j]).wait()

  def start_scatter(j, s):
    pltpu.async_copy(obuf[j].at[:, :, pl.ds(0, BBLK)],
                     out_ref.at[s, :, w], ssem[j])

  def wait_scatter(j):
    pltpu.make_async_copy(obuf[j].at[:, :, pl.ds(0, BBLK)],
                          out_ref.at[0, :, w], ssem[j]).wait()

  iota = lax.iota(jnp.int32, LANES)
  dh_q = [(iota + q * LANES) // 8 for q in range(NQ)]
  dl_q = [(iota + q * LANES) % 8 for q in range(NQ)]

  def compute(j, s):
    pq = [pos_v[s, pl.ds(q * LANES, LANES)] for q in range(NQ)]

    @pl.loop(0, BBLK, unroll=2)
    def _(b):
      bvec = jnp.broadcast_to(b, (LANES,))
      for q in range(NQ):
        v = rows[j][b, pl.ds(q * LANES, LANES)]
        plsc.store_scatter(obuf[j], [dh_q[q], dl_q[q], bvec],
                           v * SCALE + pq[q])

  for sh in range(SEQ // 8):
    pltpu.sync_copy(inp_ref.at[sh, w], idx_v.at[sh])
  pltpu.sync_copy(pos_ref, pos_v)

  for j in range(NBUF):
    start_gather(j, jnp.int32(j))

  @pl.loop(0, SEQ // NBUF)
  def _(grp):
    for j in range(NBUF):
      s = grp * NBUF + j
      wait_gather(j)

      @pl.when(s >= NBUF)
      def _():
        wait_scatter(j)

      compute(j, s)
      start_scatter(j, s)

      @pl.when(s + NBUF < SEQ)
      def _():
        start_gather(j, s + NBUF)

  for j in range(NBUF):
    wait_scatter(j)


@jax.jit
def _embed(inp4d, token_table, pos_table):
  mesh = plsc.VectorSubcoreMesh(core_axis_name="c", subcore_axis_name="s")
  run = pl.kernel(
      _body,
      out_type=jax.ShapeDtypeStruct((SEQ, DIM // 8, NW, 8, BBLK), jnp.float32),
      mesh=mesh,
      compiler_params=pltpu.CompilerParams(
          use_tc_tiling_on_sc=False, needs_layout_passes=False),
      scratch_types=[
          pltpu.VMEM((SEQ // 8, 8, BBLK), jnp.int32),
          pltpu.VMEM((SEQ, DIM), jnp.float32),
      ] + [pltpu.VMEM((BBLK, DIM), jnp.float32) for _ in range(NBUF)]
        + [pltpu.VMEM((DIM // 8, 8, BBLK + 1), jnp.float32) for _ in range(NBUF)]
        + [pltpu.SemaphoreType.DMA for _ in range(2 * NBUF)],
  )
  return run(inp4d, token_table, pos_table)


def kernel(inputs, token_table, pos_table):
  inp4d = (inputs.astype(jnp.int32).T
           .reshape(SEQ // 8, 8, NW, BBLK)
           .transpose(0, 2, 1, 3))
  out5d = _embed(inp4d, token_table, pos_table)
  out = (out5d.transpose(2, 4, 0, 1, 3)
         .reshape(BATCH, SEQ, DIM))
  return out


# speedup vs baseline: 2.7143x; 1.6193x over previous
import jax
import jax.numpy as jnp
from jax import lax
from jax.experimental import pallas as pl
from jax.experimental.pallas import tpu as pltpu
from jax.experimental.pallas import tpu_sc as plsc

SEQ = 200
DIM = 64
BATCH = 4096
NUM_CORES = 2
NUM_SUBCORES = 16
NW = NUM_CORES * NUM_SUBCORES
BBLK = BATCH // NW
NBUF = 4
LANES = 16
NQ = DIM // LANES
SCALE = 8.0


def _body(inp_ref, tok_ref, pos_ref, out_ref,
          idx_v, pos_v, rows0, rows1, rows2, rows3, ob0, ob1, ob2, ob3,
          gsem0, gsem1, gsem2, gsem3, ssem0, ssem1, ssem2, ssem3):
  rows = (rows0, rows1, rows2, rows3)
  obuf = (ob0, ob1, ob2, ob3)
  gsem = (gsem0, gsem1, gsem2, gsem3)
  ssem = (ssem0, ssem1, ssem2, ssem3)

  w = lax.axis_index("s") * NUM_CORES + lax.axis_index("c")

  def start_gather(j, s):
    sh = s // 8
    sl = s - sh * 8
    pltpu.async_copy(tok_ref.at[idx_v.at[sh, sl]], rows[j], gsem[j])

  def wait_gather(j):
    pltpu.make_async_copy(tok_ref.at[pl.ds(0, BBLK)], rows[j], gsem[j]).wait()

  def start_scatter(j, s):
    pltpu.async_copy(obuf[j].at[:, :, pl.ds(0, BBLK)],
                     out_ref.at[s, :, w], ssem[j])

  def wait_scatter(j):
    pltpu.make_async_copy(obuf[j].at[:, :, pl.ds(0, BBLK)],
                          out_ref.at[0, :, w], ssem[j]).wait()

  iota = lax.iota(jnp.int32, LANES)
  dh_q = [(iota + q * LANES) // 8 for q in range(NQ)]
  dl_q = [(iota + q * LANES) % 8 for q in range(NQ)]

  def compute(j, s):
    pq = [pos_v[s, pl.ds(q * LANES, LANES)] for q in range(NQ)]

    @plsc.parallel_loop(0, BBLK, 1, unroll=8)
    def _(b):
      bvec = jnp.broadcast_to(b, (LANES,))
      for q in range(NQ):
        v = rows[j][b, pl.ds(q * LANES, LANES)]
        plsc.store_scatter(obuf[j], [dh_q[q], dl_q[q], bvec],
                           v * SCALE + pq[q])

  for sh in range(SEQ // 8):
    pltpu.sync_copy(inp_ref.at[sh, w], idx_v.at[sh])
  pltpu.sync_copy(pos_ref, pos_v)

  for j in range(NBUF):
    start_gather(j, jnp.int32(j))

  @pl.loop(0, SEQ // NBUF)
  def _(grp):
    for j in range(NBUF):
      s = grp * NBUF + j
      wait_gather(j)

      @pl.when(s >= NBUF)
      def _():
        wait_scatter(j)

      compute(j, s)
      start_scatter(j, s)

      @pl.when(s + NBUF < SEQ)
      def _():
        start_gather(j, s + NBUF)

  for j in range(NBUF):
    wait_scatter(j)


@jax.jit
def _embed(inp4d, token_table, pos_table):
  mesh = plsc.VectorSubcoreMesh(core_axis_name="c", subcore_axis_name="s")
  run = pl.kernel(
      _body,
      out_type=jax.ShapeDtypeStruct((SEQ, DIM // 8, NW, 8, BBLK), jnp.float32),
      mesh=mesh,
      compiler_params=pltpu.CompilerParams(
          use_tc_tiling_on_sc=False, needs_layout_passes=False),
      scratch_types=[
          pltpu.VMEM((SEQ // 8, 8, BBLK), jnp.int32),
          pltpu.VMEM((SEQ, DIM), jnp.float32),
      ] + [pltpu.VMEM((BBLK, DIM), jnp.float32) for _ in range(NBUF)]
        + [pltpu.VMEM((DIM // 8, 8, BBLK + 1), jnp.float32) for _ in range(NBUF)]
        + [pltpu.SemaphoreType.DMA for _ in range(2 * NBUF)],
  )
  return run(inp4d, token_table, pos_table)


def kernel(inputs, token_table, pos_table):
  inp4d = (inputs.astype(jnp.int32).T
           .reshape(SEQ // 8, 8, NW, BBLK)
           .transpose(0, 2, 1, 3))
  out5d = _embed(inp4d, token_table, pos_table)
  out = (out5d.transpose(2, 4, 0, 1, 3)
         .reshape(BATCH, SEQ, DIM))
  return out
